# baseline (speedup 1.0000x reference)
import jax
import jax.numpy as jnp
from jax.experimental import pallas as pl


def _fused_t(tt_ref, xs_ref, mk_ref, tp_ref, nva_ref,
             cc_ref, sel_ref, selp_ref, band_ref,
             wcatT_ref, w1aT_ref, w1bT_ref, w2T_ref,
             w3_ref, b3_ref, out_ref):
    HID = cc_ref.shape[0]
    G, W = sel_ref.shape
    L = W // G
    LP = selp_ref.shape[1] // G
    f32 = jnp.float32

    tt = tt_ref[...].reshape(1, W)
    xs = xs_ref[...].reshape(1, W)
    mk = mk_ref[...].reshape(1, W)

    cc = cc_ref[...]
    wall = cc[:, 0:1]
    ball = cc[:, 1:2]
    c3 = cc[:, 2:3]
    gcnb = cc[:, 3:4]
    b1 = cc[:, 4:5]
    b2 = cc[:, 5:6]

    def poly(r):
        r2 = r * r
        return r * (1.0 + r2 * c3)

    bf16 = jnp.bfloat16
    lin = tt * wall + ball
    te = poly(lin)

    enc = jnp.dot(nva_ref[0],
                  jnp.concatenate([sel_ref[...], xs], axis=0),
                  preferred_element_type=f32)
    Hb = jax.nn.relu((enc + te).astype(bf16))

    mh = mk * Hb
    band = band_ref[...]
    lr = jnp.concatenate(
        [jnp.dot(mh[:, g * L:(g + 1) * L], band, preferred_element_type=f32)
         for g in range(G)], axis=1).astype(bf16)
    mk32 = mk.astype(f32)
    z1c = jnp.zeros((1, 1), f32)
    ml = jnp.concatenate([z1c, mk32[:, :-1]], axis=1)
    mr = jnp.concatenate([mk32[:, 1:], z1c], axis=1)
    ii = jax.lax.broadcasted_iota(jnp.int32, (1, W), 1)
    tpos = jax.lax.rem(ii, L)
    ml = jnp.where(tpos == 0, 0.0, ml)
    mr = jnp.where(tpos == L - 1, 0.0, mr)
    deg = mk32 * (ml + mr)
    s = mk32 / (deg + 1e-6)

    aggn = s.astype(bf16) * lr
    h = jax.nn.relu(
        jnp.dot(wcatT_ref[...], jnp.concatenate([Hb, aggn], axis=0),
                preferred_element_type=f32)
        + gcnb)
    pooled = jnp.concatenate(
        [jnp.sum(h[:, g * L:(g + 1) * L], axis=1, keepdims=True)
         for g in range(G)], axis=1) * (1.0 / L)

    WP = G * LP
    tp = tp_ref[...].reshape(1, WP)
    lin2 = tp * wall + ball
    tep = poly(lin2)
    pa = jnp.dot(w1aT_ref[...], pooled,
                 preferred_element_type=f32) + b1
    z1 = jax.nn.relu(
        jnp.dot(jnp.concatenate([pa, w1bT_ref[...]], axis=1),
                jnp.concatenate([selp_ref[...], tep], axis=0),
                preferred_element_type=f32))
    z2 = jax.nn.relu(jnp.dot(w2T_ref[...], z1, preferred_element_type=f32)
                     + b2)
    o = jnp.dot(w3_ref[...], z2, preferred_element_type=f32) + b3_ref[...]
    out_ref[...] = o.reshape(1, 1, WP)


def kernel(time_steps_to_predict, X, truth_time_steps, mask,
           te_scale_w, te_scale_b, te_per_w, te_per_b,
           obs_w, obs_b, nodevec,
           gcn_w_self, gcn_w_nei, gcn_b,
           dec_w1, dec_b1, dec_w2, dec_b2, dec_w3, dec_b3):
    B, M, L, N = X.shape
    HID = nodevec.shape[1]
    LP = time_steps_to_predict.shape[-1]
    S = B * N
    f32 = jnp.float32

    G = 160
    NB = S // G
    W = G * L

    bf16 = jnp.bfloat16
    xs = X[:, 0].transpose(0, 2, 1).reshape(NB, 1, W).astype(bf16)
    tts = truth_time_steps[:, 0].transpose(0, 2, 1).reshape(NB, 1, W)
    mks = mask[:, 0].transpose(0, 2, 1).reshape(NB, 1, W).astype(bf16)
    tps = jnp.broadcast_to(time_steps_to_predict[:, None, :],
                           (B, N, LP)).reshape(NB, 1, G * LP)

    nvT = (jnp.broadcast_to(nodevec[None], (B, N, HID)).reshape(S, HID)
           + obs_b[None, :]).T
    nva = jnp.concatenate(
        [nvT.reshape(HID, NB, G).transpose(1, 0, 2),
         jnp.broadcast_to(obs_w.reshape(1, HID, 1), (NB, HID, 1))],
        axis=2).astype(bf16)

    wall = jnp.concatenate([te_scale_w.reshape(1),
                            te_per_w.reshape(HID - 1)])
    ball = jnp.concatenate([te_scale_b.reshape(1),
                            te_per_b.reshape(HID - 1)])
    lane0 = (jnp.arange(HID) == 0)
    c3 = jnp.where(lane0, 0.0, -0.16605)
    cc = jnp.stack([wall, ball, c3, gcn_b, dec_b1, dec_b2],
                   axis=1).astype(f32)
    wcatT = jnp.concatenate([gcn_w_self.T, gcn_w_nei.T],
                            axis=1).astype(bf16)

    gidx = jnp.arange(W) // L
    sel = (gidx[None, :] == jnp.arange(G)[:, None]).astype(bf16)
    gidxp = jnp.arange(G * LP) // LP
    selp = (gidxp[None, :] == jnp.arange(G)[:, None]).astype(f32)
    t_i = jnp.arange(L)
    band = (jnp.abs(t_i[:, None] - t_i[None, :]) == 1).astype(jnp.bfloat16)

    def sblk(i):
        return (i, 0, 0)

    def wblk2(i):
        return (0, 0)

    out = pl.pallas_call(
        _fused_t,
        grid=(NB,),
        in_specs=[
            pl.BlockSpec((1, 1, W), sblk),
            pl.BlockSpec((1, 1, W), sblk),
            pl.BlockSpec((1, 1, W), sblk),
            pl.BlockSpec((1, 1, G * LP), sblk),
            pl.BlockSpec((1, HID, G + 1), lambda i: (i, 0, 0)),
            pl.BlockSpec((HID, 6), wblk2),
            pl.BlockSpec((G, W), wblk2),
            pl.BlockSpec((G, G * LP), wblk2),
            pl.BlockSpec((L, L), wblk2),
            pl.BlockSpec((HID, 2 * HID), wblk2),
            pl.BlockSpec((HID, HID), wblk2),
            pl.BlockSpec((HID, HID), wblk2),
            pl.BlockSpec((HID, HID), wblk2),
            pl.BlockSpec((1, HID), wblk2),
            pl.BlockSpec((1, 1), wblk2),
        ],
        out_specs=pl.BlockSpec((1, 1, G * LP), sblk),
        out_shape=jax.ShapeDtypeStruct((NB, 1, G * LP), f32),
    )(tts, xs, mks, tps, nva, cc, sel, selp, band,
      wcatT, dec_w1[:HID].T, dec_w1[HID:].T, dec_w2.T,
      dec_w3.reshape(1, HID), dec_b3.reshape(1, 1))

    z = out.reshape(B, N, LP)
    return jnp.transpose(z, (0, 2, 1))[None]


# speedup vs baseline: 1.1044x; 1.1044x over previous
import jax
import jax.numpy as jnp
from jax.experimental import pallas as pl


def _fused_t(tt_ref, xs_ref, mk_ref, tp_ref, nva_ref,
             cc_ref, sel_ref, selp_ref, band_ref,
             wcatT_ref, w1aT_ref, w1bT_ref, w2T_ref,
             w3_ref, b3_ref, out_ref):
    HID = cc_ref.shape[0]
    G, W = sel_ref.shape
    L = W // G
    LP = selp_ref.shape[1] // G
    f32 = jnp.float32

    tt = tt_ref[...].reshape(1, W)
    xs = xs_ref[...].reshape(1, W)
    mk = mk_ref[...].reshape(1, W)

    cc = cc_ref[...]
    wall = cc[:, 0:1]
    ball = cc[:, 1:2]
    c3 = cc[:, 2:3]
    gcnb = cc[:, 3:4]
    b1 = cc[:, 4:5]
    b2 = cc[:, 5:6]

    def poly(r):
        r2 = r * r
        return r * (1.0 + r2 * c3)

    bf16 = jnp.bfloat16
    lin_b = tt.astype(bf16) * wall.astype(bf16) + ball.astype(bf16)
    r2b = lin_b * lin_b
    te = lin_b * (1.0 + r2b * c3.astype(bf16))

    enc = jnp.dot(nva_ref[0],
                  jnp.concatenate([sel_ref[...], xs], axis=0),
                  preferred_element_type=f32)
    Hb = jax.nn.relu(enc.astype(bf16) + te)

    mh = mk * Hb
    band = band_ref[...]
    lr = jnp.concatenate(
        [jnp.dot(mh[:, g * L:(g + 1) * L], band, preferred_element_type=f32)
         for g in range(G)], axis=1).astype(bf16)
    mk32 = mk.astype(f32)
    z1c = jnp.zeros((1, 1), f32)
    ml = jnp.concatenate([z1c, mk32[:, :-1]], axis=1)
    mr = jnp.concatenate([mk32[:, 1:], z1c], axis=1)
    ii = jax.lax.broadcasted_iota(jnp.int32, (1, W), 1)
    tpos = jax.lax.rem(ii, L)
    ml = jnp.where(tpos == 0, 0.0, ml)
    mr = jnp.where(tpos == L - 1, 0.0, mr)
    deg = mk32 * (ml + mr)
    s = mk32 / (deg + 1e-6)

    aggn = s.astype(bf16) * lr
    h = jax.nn.relu(
        jnp.dot(wcatT_ref[...], jnp.concatenate([Hb, aggn], axis=0),
                preferred_element_type=f32)
        + gcnb)
    pooled = jnp.concatenate(
        [jnp.sum(h[:, g * L:(g + 1) * L], axis=1, keepdims=True)
         for g in range(G)], axis=1) * (1.0 / L)

    WP = G * LP
    tp = tp_ref[...].reshape(1, WP)
    lin2 = tp * wall + ball
    tep = poly(lin2)
    pa = jnp.dot(w1aT_ref[...], pooled,
                 preferred_element_type=f32) + b1
    z1 = jax.nn.relu(
        jnp.dot(jnp.concatenate([pa, w1bT_ref[...]], axis=1),
                jnp.concatenate([selp_ref[...], tep], axis=0),
                preferred_element_type=f32))
    z2 = jax.nn.relu(jnp.dot(w2T_ref[...], z1, preferred_element_type=f32)
                     + b2)
    o = jnp.dot(w3_ref[...], z2, preferred_element_type=f32) + b3_ref[...]
    out_ref[...] = o.reshape(1, 1, WP)


def kernel(time_steps_to_predict, X, truth_time_steps, mask,
           te_scale_w, te_scale_b, te_per_w, te_per_b,
           obs_w, obs_b, nodevec,
           gcn_w_self, gcn_w_nei, gcn_b,
           dec_w1, dec_b1, dec_w2, dec_b2, dec_w3, dec_b3):
    B, M, L, N = X.shape
    HID = nodevec.shape[1]
    LP = time_steps_to_predict.shape[-1]
    S = B * N
    f32 = jnp.float32

    G = 160
    NB = S // G
    W = G * L

    bf16 = jnp.bfloat16
    xs = X[:, 0].transpose(0, 2, 1).reshape(NB, 1, W).astype(bf16)
    tts = truth_time_steps[:, 0].transpose(0, 2, 1).reshape(NB, 1, W)
    mks = mask[:, 0].transpose(0, 2, 1).reshape(NB, 1, W).astype(bf16)
    tps = jnp.broadcast_to(time_steps_to_predict[:, None, :],
                           (B, N, LP)).reshape(NB, 1, G * LP)

    nvT = (jnp.broadcast_to(nodevec[None], (B, N, HID)).reshape(S, HID)
           + obs_b[None, :]).T
    nva = jnp.concatenate(
        [nvT.reshape(HID, NB, G).transpose(1, 0, 2),
         jnp.broadcast_to(obs_w.reshape(1, HID, 1), (NB, HID, 1))],
        axis=2).astype(bf16)

    wall = jnp.concatenate([te_scale_w.reshape(1),
                            te_per_w.reshape(HID - 1)])
    ball = jnp.concatenate([te_scale_b.reshape(1),
                            te_per_b.reshape(HID - 1)])
    lane0 = (jnp.arange(HID) == 0)
    c3 = jnp.where(lane0, 0.0, -0.16605)
    cc = jnp.stack([wall, ball, c3, gcn_b, dec_b1, dec_b2],
                   axis=1).astype(f32)
    wcatT = jnp.concatenate([gcn_w_self.T, gcn_w_nei.T],
                            axis=1).astype(bf16)

    gidx = jnp.arange(W) // L
    sel = (gidx[None, :] == jnp.arange(G)[:, None]).astype(bf16)
    gidxp = jnp.arange(G * LP) // LP
    selp = (gidxp[None, :] == jnp.arange(G)[:, None]).astype(f32)
    t_i = jnp.arange(L)
    band = (jnp.abs(t_i[:, None] - t_i[None, :]) == 1).astype(jnp.bfloat16)

    def sblk(i):
        return (i, 0, 0)

    def wblk2(i):
        return (0, 0)

    out = pl.pallas_call(
        _fused_t,
        grid=(NB,),
        in_specs=[
            pl.BlockSpec((1, 1, W), sblk),
            pl.BlockSpec((1, 1, W), sblk),
            pl.BlockSpec((1, 1, W), sblk),
            pl.BlockSpec((1, 1, G * LP), sblk),
            pl.BlockSpec((1, HID, G + 1), lambda i: (i, 0, 0)),
            pl.BlockSpec((HID, 6), wblk2),
            pl.BlockSpec((G, W), wblk2),
            pl.BlockSpec((G, G * LP), wblk2),
            pl.BlockSpec((L, L), wblk2),
            pl.BlockSpec((HID, 2 * HID), wblk2),
            pl.BlockSpec((HID, HID), wblk2),
            pl.BlockSpec((HID, HID), wblk2),
            pl.BlockSpec((HID, HID), wblk2),
            pl.BlockSpec((1, HID), wblk2),
            pl.BlockSpec((1, 1), wblk2),
        ],
        out_specs=pl.BlockSpec((1, 1, G * LP), sblk),
        out_shape=jax.ShapeDtypeStruct((NB, 1, G * LP), f32),
    )(tts, xs, mks, tps, nva, cc, sel, selp, band,
      wcatT, dec_w1[:HID].T, dec_w1[HID:].T, dec_w2.T,
      dec_w3.reshape(1, HID), dec_b3.reshape(1, 1))

    z = out.reshape(B, N, LP)
    return jnp.transpose(z, (0, 2, 1))[None]
